# baseline (speedup 1.0000x reference)
import jax
import jax.numpy as jnp
from jax import lax
from jax.experimental import pallas as pl
from jax.experimental.pallas import tpu as pltpu
from jax.experimental.pallas import tpu_sc as plsc

H = 1024
EPS = 1e-6
NC, NS = 2, 16
NW = NC * NS
L = 16
NSL = H // L
C = 8
NG = 8
NSB = 4


def _rsqrt_vec(v):
    iv = plsc.bitcast(v, jnp.int32)
    magic = jnp.full((L,), 0x5F3759DF, jnp.int32)
    y = plsc.bitcast(magic - (iv >> 1), jnp.float32)
    h = v * 0.5
    for _ in range(2):
        y = y * (1.5 - h * y * y)
    return y


def _make_body(n_rows, general):
    rpw = n_rows // NW
    nchunk = rpw // C

    def body(ids_hbm, table_hbm, scale_hbm, bias_hbm, lnw_hbm, lnb_hbm,
             out_hbm, idxv, sref, bref, wref, lbref, statb, ibuf, obuf,
             *sems):
        gsem = list(sems[:NG])
        ssem = list(sems[NG:])
        wid = lax.axis_index("c") * NS + lax.axis_index("s")
        base = wid * rpw

        pltpu.sync_copy(ids_hbm.at[pl.ds(base, rpw)], idxv)
        if general:
            pltpu.sync_copy(scale_hbm, sref)
            pltpu.sync_copy(bias_hbm, bref)
            pltpu.sync_copy(lnw_hbm, wref)
            pltpu.sync_copy(lnb_hbm, lbref)

        def gather_desc(c, b):
            return pltpu.make_async_copy(
                table_hbm.at[idxv.at[pl.ds(c * C, C)]], ibuf.at[b], gsem[b])

        def scatter_desc(c, b):
            return pltpu.make_async_copy(
                obuf.at[b], out_hbm.at[pl.ds(base + c * C, C)], ssem[b])

        iota = lax.iota(jnp.int32, L)
        colbase = iota * L

        def compute(ib, ob):
            zero = jnp.zeros((L,), jnp.float32)

            if general:
                @plsc.parallel_loop(0, NSL, unroll=2,
                                    carry=((zero,) * C, (zero,) * C))
                def p1(j, carry):
                    a1, a2 = carry
                    sl = pl.ds(j * L, L)
                    s = sref[sl]
                    bb = bref[sl]
                    n1 = []
                    n2 = []
                    for r in range(C):
                        x = ib[r, sl]
                        e = x * s + bb
                        ib[r, sl] = e
                        n1.append(a1[r] + e)
                        n2.append(a2[r] + e * e)
                    return (tuple(n1), tuple(n2))
            else:
                @plsc.parallel_loop(0, NSL, unroll=2,
                                    carry=((zero,) * C, (zero,) * C))
                def p1(j, carry):
                    a1, a2 = carry
                    sl = pl.ds(j * L, L)
                    n1 = []
                    n2 = []
                    for r in range(C):
                        x = ib[r, sl]
                        n1.append(a1[r] + x)
                        n2.append(a2[r] + x * x)
                    return (tuple(n1), tuple(n2))

            a1, a2 = p1

            for r in range(C):
                statb[r] = a1[r]
                statb[C + r] = a2[r]
            s1 = zero
            s2 = zero
            for c in range(L):
                s1 = s1 + plsc.load_gather(statb, [iota, jnp.full((L,), c, jnp.int32)])
                s2 = s2 + plsc.load_gather(statb, [iota + C, jnp.full((L,), c, jnp.int32)])
            mean = s1 * (1.0 / H)
            var = s2 * (1.0 / H) - mean * mean
            rv = _rsqrt_vec(var + EPS)
            mus = []
            rvs = []
            for r in range(C):
                mus.append(jnp.full((L,), mean[r], jnp.float32))
                rvs.append(jnp.full((L,), rv[r], jnp.float32))

            if general:
                @plsc.parallel_loop(0, NSL, unroll=2)
                def p2(j):
                    sl = pl.ds(j * L, L)
                    w = wref[sl]
                    lb = lbref[sl]
                    for r in range(C):
                        e = ib[r, sl]
                        ob[r, sl] = (e - mus[r]) * rvs[r] * w + lb
            else:
                @plsc.parallel_loop(0, NSL, unroll=2)
                def p2(j):
                    sl = pl.ds(j * L, L)
                    for r in range(C):
                        x = ib[r, sl]
                        ob[r, sl] = (x - mus[r]) * rvs[r]

        for b in range(NG):
            gather_desc(b, b).start()

        def step(c, b):
            ob = b % NSB
            @pl.when(c >= NSB)
            def _():
                scatter_desc(c - NSB, ob).wait()
            gather_desc(c, b).wait()
            compute(ibuf.at[b], obuf.at[ob])

            @pl.when(c + NG < nchunk)
            def _():
                gather_desc(c + NG, b).start()
            scatter_desc(c, ob).start()

        def outer(g, carry):
            for b in range(NG):
                step(g * NG + b, b)
            return carry

        lax.fori_loop(0, nchunk // NG, outer, 0)

        for b in range(NSB):
            c_last = nchunk - NSB + b
            scatter_desc(c_last, c_last % NSB).wait()

    return body


def _make_run(n_rows, general):
    mesh = plsc.VectorSubcoreMesh(core_axis_name="c", subcore_axis_name="s",
                                  num_cores=NC, num_subcores=NS)
    return pl.kernel(
        _make_body(n_rows, general),
        out_type=jax.ShapeDtypeStruct((n_rows, H), jnp.float32),
        mesh=mesh,
        scratch_types=[
            pltpu.VMEM((n_rows // NW,), jnp.int32),
            pltpu.VMEM((H,), jnp.float32),
            pltpu.VMEM((H,), jnp.float32),
            pltpu.VMEM((H,), jnp.float32),
            pltpu.VMEM((H,), jnp.float32),
            pltpu.VMEM((2 * C, L), jnp.float32),
            pltpu.VMEM((NG, C, H), jnp.float32),
            pltpu.VMEM((NSB, C, H), jnp.float32),
        ] + [pltpu.SemaphoreType.DMA] * (NG + NSB),
        compiler_params=pltpu.CompilerParams(needs_layout_passes=False),
        name="yv_token_embedding_sc" + ("_gen" if general else "_fast"),
    )


def kernel(input_ids, table, scale, bias, ln_weight, ln_bias):
    bsz, seq = input_ids.shape
    ids = input_ids.reshape(-1).astype(jnp.int32)
    n_rows = bsz * seq
    trivial = (jnp.all(scale == 1.0) & jnp.all(bias == 0.0)
               & jnp.all(ln_weight == 1.0) & jnp.all(ln_bias == 0.0))
    run_gen = _make_run(n_rows, True)
    run_fast = _make_run(n_rows, False)
    out = lax.cond(
        trivial,
        lambda: run_fast(ids, table, scale, bias, ln_weight, ln_bias),
        lambda: run_gen(ids, table, scale, bias, ln_weight, ln_bias),
    )
    return out.reshape(bsz, seq, H)


# speedup vs baseline: 1.0545x; 1.0545x over previous
import jax
import jax.numpy as jnp
from jax import lax
from jax.experimental import pallas as pl
from jax.experimental.pallas import tpu as pltpu
from jax.experimental.pallas import tpu_sc as plsc

H = 1024
EPS = 1e-6
NC, NS = 2, 16
NW = NC * NS
L = 16
NSL = H // L
C = 8
NG = 4
NSB = 4


def _rsqrt_vec(v):
    iv = plsc.bitcast(v, jnp.int32)
    magic = jnp.full((L,), 0x5F3759DF, jnp.int32)
    y = plsc.bitcast(magic - (iv >> 1), jnp.float32)
    h = v * 0.5
    for _ in range(2):
        y = y * (1.5 - h * y * y)
    return y


def _make_body(n_rows, general):
    rpw = n_rows // NW
    nchunk = rpw // C

    def body(ids_hbm, table_hbm, scale_hbm, bias_hbm, lnw_hbm, lnb_hbm,
             out_hbm, idxv, sref, bref, wref, lbref, statb, ibuf, obuf,
             *sems):
        gsem = list(sems[:NG])
        ssem = list(sems[NG:])
        wid = lax.axis_index("c") * NS + lax.axis_index("s")
        base = wid * rpw

        pltpu.sync_copy(ids_hbm.at[pl.ds(base, rpw)], idxv)
        if general:
            pltpu.sync_copy(scale_hbm, sref)
            pltpu.sync_copy(bias_hbm, bref)
            pltpu.sync_copy(lnw_hbm, wref)
            pltpu.sync_copy(lnb_hbm, lbref)

        def gather_desc(c, b):
            return pltpu.make_async_copy(
                table_hbm.at[idxv.at[pl.ds(c * C, C)]], ibuf.at[b], gsem[b])

        def scatter_desc(c, b):
            return pltpu.make_async_copy(
                obuf.at[b], out_hbm.at[pl.ds(base + c * C, C)], ssem[b])

        iota = lax.iota(jnp.int32, L)
        colbase = iota * L

        def compute(ib, ob):
            zero = jnp.zeros((L,), jnp.float32)

            if general:
                @plsc.parallel_loop(0, NSL, unroll=2,
                                    carry=((zero,) * C, (zero,) * C))
                def p1(j, carry):
                    a1, a2 = carry
                    sl = pl.ds(j * L, L)
                    s = sref[sl]
                    bb = bref[sl]
                    n1 = []
                    n2 = []
                    for r in range(C):
                        x = ib[r, sl]
                        e = x * s + bb
                        ib[r, sl] = e
                        n1.append(a1[r] + e)
                        n2.append(a2[r] + e * e)
                    return (tuple(n1), tuple(n2))
            else:
                @plsc.parallel_loop(0, NSL, unroll=2,
                                    carry=((zero,) * C, (zero,) * C))
                def p1(j, carry):
                    a1, a2 = carry
                    sl = pl.ds(j * L, L)
                    n1 = []
                    n2 = []
                    for r in range(C):
                        x = ib[r, sl]
                        n1.append(a1[r] + x)
                        n2.append(a2[r] + x * x)
                    return (tuple(n1), tuple(n2))

            a1, a2 = p1

            for r in range(C):
                statb[r] = a1[r]
                statb[C + r] = a2[r]
            s1 = zero
            s2 = zero
            for c in range(L):
                s1 = s1 + plsc.load_gather(statb, [iota, jnp.full((L,), c, jnp.int32)])
                s2 = s2 + plsc.load_gather(statb, [iota + C, jnp.full((L,), c, jnp.int32)])
            mean = s1 * (1.0 / H)
            var = s2 * (1.0 / H) - mean * mean
            rv = _rsqrt_vec(var + EPS)
            mus = []
            rvs = []
            for r in range(C):
                mus.append(jnp.full((L,), mean[r], jnp.float32))
                rvs.append(jnp.full((L,), rv[r], jnp.float32))

            if general:
                @plsc.parallel_loop(0, NSL, unroll=2)
                def p2(j):
                    sl = pl.ds(j * L, L)
                    w = wref[sl]
                    lb = lbref[sl]
                    for r in range(C):
                        e = ib[r, sl]
                        ob[r, sl] = (e - mus[r]) * rvs[r] * w + lb
            else:
                @plsc.parallel_loop(0, NSL, unroll=2)
                def p2(j):
                    sl = pl.ds(j * L, L)
                    for r in range(C):
                        x = ib[r, sl]
                        ob[r, sl] = (x - mus[r]) * rvs[r]

        for b in range(NG):
            gather_desc(b, b).start()

        def step(c, b):
            ob = b % NSB
            @pl.when(c >= NSB)
            def _():
                scatter_desc(c - NSB, ob).wait()
            gather_desc(c, b).wait()
            compute(ibuf.at[b], obuf.at[ob])

            @pl.when(c + NG < nchunk)
            def _():
                gather_desc(c + NG, b).start()
            scatter_desc(c, ob).start()

        def outer(g, carry):
            for b in range(NG):
                step(g * NG + b, b)
            return carry

        lax.fori_loop(0, nchunk // NG, outer, 0)

        for b in range(NSB):
            c_last = nchunk - NSB + b
            scatter_desc(c_last, c_last % NSB).wait()

    return body


def _make_run(n_rows, general):
    mesh = plsc.VectorSubcoreMesh(core_axis_name="c", subcore_axis_name="s",
                                  num_cores=NC, num_subcores=NS)
    return pl.kernel(
        _make_body(n_rows, general),
        out_type=jax.ShapeDtypeStruct((n_rows, H), jnp.float32),
        mesh=mesh,
        scratch_types=[
            pltpu.VMEM((n_rows // NW,), jnp.int32),
            pltpu.VMEM((H,), jnp.float32),
            pltpu.VMEM((H,), jnp.float32),
            pltpu.VMEM((H,), jnp.float32),
            pltpu.VMEM((H,), jnp.float32),
            pltpu.VMEM((2 * C, L), jnp.float32),
            pltpu.VMEM((NG, C, H), jnp.float32),
            pltpu.VMEM((NSB, C, H), jnp.float32),
        ] + [pltpu.SemaphoreType.DMA] * (NG + NSB),
        compiler_params=pltpu.CompilerParams(needs_layout_passes=False),
        name="yv_token_embedding_sc" + ("_gen" if general else "_fast"),
    )


def kernel(input_ids, table, scale, bias, ln_weight, ln_bias):
    bsz, seq = input_ids.shape
    ids = input_ids.reshape(-1).astype(jnp.int32)
    n_rows = bsz * seq
    trivial = (jnp.all(scale == 1.0) & jnp.all(bias == 0.0)
               & jnp.all(ln_weight == 1.0) & jnp.all(ln_bias == 0.0))
    run_gen = _make_run(n_rows, True)
    run_fast = _make_run(n_rows, False)
    out = lax.cond(
        trivial,
        lambda: run_fast(ids, table, scale, bias, ln_weight, ln_bias),
        lambda: run_gen(ids, table, scale, bias, ln_weight, ln_bias),
    )
    return out.reshape(bsz, seq, H)
